# baseline (speedup 1.0000x reference)
import functools

import jax
import jax.numpy as jnp
from jax import lax
from jax.experimental import pallas as pl
from jax.experimental.pallas import tpu as pltpu
from jax.experimental.pallas import tpu_sc as plsc


def _splat(tvec, idx):
    return lax.gather(
        tvec,
        jnp.full((16, 1), idx, jnp.int32),
        lax.GatherDimensionNumbers(
            offset_dims=(), collapsed_slice_dims=(0,), start_index_map=(0,)
        ),
        (1,),
        mode=lax.GatherScatterMode.PROMISE_IN_BOUNDS,
    )


def _make_sc_add(L, D, B):
    NC = 2
    NW = 32
    OCT = D // 8
    UNITS = L * OCT
    UPW = UNITS // NW
    mesh = plsc.VectorSubcoreMesh(core_axis_name="c", subcore_axis_name="s")

    @functools.partial(
        pl.kernel,
        mesh=mesh,
        out_type=jax.ShapeDtypeStruct((L, D, B), jnp.float32),
        scratch_types=[
            pltpu.VMEM((3, 8, B), jnp.float32),
            pltpu.VMEM((3, 16), jnp.float32),
            pltpu.SemaphoreType.DMA,
            pltpu.SemaphoreType.DMA,
        ],
    )
    def sc_add(x_hbm, t_hbm, out_hbm, buf, tv, isem, osem):
        wid = lax.axis_index("s") * NC + lax.axis_index("c")
        u0 = wid * UPW

        def start_in(u, slot):
            l = u // OCT
            a = u % OCT
            pltpu.sync_copy(
                t_hbm.at[pl.ds(l * D + 8 * a, 8)], tv.at[slot, pl.ds(0, 8)]
            )
            pltpu.async_copy(x_hbm.at[l, pl.ds(8 * a, 8), :], buf.at[slot], isem)

        def wait_one_in(slot):
            pltpu.make_async_copy(
                x_hbm.at[0, pl.ds(0, 8), :], buf.at[slot], isem
            ).wait()

        def drain_one_out(slot):
            pltpu.make_async_copy(
                buf.at[slot], out_hbm.at[0, pl.ds(0, 8), :], osem
            ).wait()

        start_in(u0, 0)
        start_in(u0 + 1, 1)

        def unit(j, carry):
            u = u0 + j
            slot = lax.rem(j, 3)
            wait_one_in(slot)
            tvec = tv[slot]
            vals = [_splat(tvec, r) for r in range(8)]

            def col(c, _):
                base = c * 16
                for r in range(8):
                    sl = pl.ds(base, 16)
                    buf[slot, r, sl] = buf[slot, r, sl] + vals[r]
                return _

            lax.fori_loop(0, B // 16, col, 0)
            l = u // OCT
            a = u % OCT
            pltpu.async_copy(buf.at[slot], out_hbm.at[l, pl.ds(8 * a, 8), :], osem)

            @pl.when(j + 2 < UPW)
            def _():
                @pl.when(j >= 1)
                def _():
                    drain_one_out(slot)
                start_in(u + 2, lax.rem(j + 2, 3))

            return carry

        lax.fori_loop(0, UPW, unit, 0)
        drain_one_out(0)
        drain_one_out(1)
        drain_one_out(2)

    return sc_add


def kernel(x, pos_table):
    B, L, D = x.shape
    xt = x.transpose(1, 2, 0)
    tflat = pos_table.reshape(-1)
    out_t = _make_sc_add(L, D, B)(xt, tflat)
    return out_t.transpose(2, 0, 1)
